# baseline (speedup 1.0000x reference)
import functools

import jax
import jax.numpy as jnp
from jax import lax
from jax.experimental import pallas as pl
from jax.experimental.pallas import tpu as pltpu
from jax.experimental.pallas import tpu_sc as plsc

BATCH = 16384
PF = 64
NC = 2
NS = 16
NW = NC * NS
B_PER_W = BATCH // NW
CHUNK = 128
N_CHUNKS = B_PER_W // CHUNK
GROUPS_PER_CHUNK = CHUNK // 16

TR_COLS = 1024
HALF_BLOCKS = 49
HALF = HALF_BLOCKS * TR_COLS
TR_GRID = 25


def _tr_body(ua1, ua2, ub1, ub2, ia1, ia2, ib1, ib2, uo_ref, io_ref):
    uo_ref[pl.ds(0, TR_COLS), :] = jnp.concatenate([ua1[...], ub1[...]], axis=0).T
    uo_ref[pl.ds(TR_COLS, TR_COLS), :] = jnp.concatenate([ua2[...], ub2[...]], axis=0).T
    io_ref[pl.ds(0, TR_COLS), :] = jnp.concatenate([ia1[...], ib1[...]], axis=0).T
    io_ref[pl.ds(TR_COLS, TR_COLS), :] = jnp.concatenate([ia2[...], ib2[...]], axis=0).T


def _transpose_tables(utT, itT):
    specs = [
        pl.BlockSpec((PF, TR_COLS), lambda i: (0, 2 * i)),
        pl.BlockSpec((PF, TR_COLS),
                     lambda i: (0, jnp.minimum(2 * i + 1, 2 * TR_GRID - 2))),
        pl.BlockSpec((PF, TR_COLS),
                     lambda i: (0, HALF_BLOCKS + 2 * i)),
        pl.BlockSpec((PF, TR_COLS),
                     lambda i: (0, jnp.minimum(HALF_BLOCKS + 2 * i + 1,
                                               2 * HALF_BLOCKS - 1))),
    ]
    out_sds = jax.ShapeDtypeStruct((2 * TR_GRID * TR_COLS, 128), jnp.float32)
    return pl.pallas_call(
        _tr_body,
        grid=(TR_GRID,),
        in_specs=specs + specs,
        out_specs=[pl.BlockSpec((2 * TR_COLS, 128), lambda i: (i, 0))] * 2,
        out_shape=[out_sds, out_sds],
    )(utT, utT, utT, utT, itT, itT, itT, itT)


def _sc_gmf_body(uid_hbm, iid_hbm, ut_hbm, it_hbm, w_hbm, b_hbm, out_hbm,
                 uidx_v, iidx_v, usup_v, isup_v, u_chunks, v_chunks,
                 w_v, b_v, out_v, t_v, idx_sem, sem):
    wid = lax.axis_index("s") * NC + lax.axis_index("c")
    base = wid * B_PER_W

    idx_cps = []
    for c in range(N_CHUNKS):
        idx_cps.append(pltpu.async_copy(
            uid_hbm.at[pl.ds(base + c * CHUNK, CHUNK)], uidx_v.at[c], idx_sem))
        idx_cps.append(pltpu.async_copy(
            iid_hbm.at[pl.ds(base + c * CHUNK, CHUNK)], iidx_v.at[c], idx_sem))
    idx_cps.append(pltpu.async_copy(w_hbm, w_v, idx_sem))
    idx_cps.append(pltpu.async_copy(b_hbm, b_v, idx_sem))
    for cp in idx_cps:
        cp.wait()

    for c in range(N_CHUNKS):
        for k in range(CHUNK // 16):
            sl = pl.ds(k * 16, 16)
            uv = uidx_v[c, sl]
            iv = iidx_v[c, sl]
            usup_v[c, sl] = uv - jnp.where(uv >= HALF, HALF, 0)
            isup_v[c, sl] = iv - jnp.where(iv >= HALF, HALF, 0)

    bias = b_v[...][0]
    w_chunks = [w_v[pl.ds(c * 16, 16)] for c in range(PF // 16)]
    lanes = lax.iota(jnp.int32, 16)

    cps = [None] * N_CHUNKS

    SUB = 32

    def fire(c):
        buf = c % 2
        cp = []
        for k in range(CHUNK // SUB):
            sl = pl.ds(k * SUB, SUB)
            cp.append(pltpu.async_copy(
                ut_hbm.at[usup_v.at[c, sl]], u_chunks.at[buf, sl], sem))
            cp.append(pltpu.async_copy(
                it_hbm.at[isup_v.at[c, sl]], v_chunks.at[buf, sl], sem))
        cps[c] = cp

    fire(0)
    for c in range(N_CHUNKS):
        if c + 1 < N_CHUNKS:
            fire(c + 1)
        for cp in cps[c]:
            cp.wait()
        u_chunk = u_chunks.at[c % 2]
        v_chunk = v_chunks.at[c % 2]

        def group_body(g, carry, c=c, u_chunk=u_chunk, v_chunk=v_chunk):
            sl = pl.ds(g * 16, 16)
            ucolv = jnp.where(uidx_v[c, sl] >= HALF, PF, 0)
            icolv = jnp.where(iidx_v[c, sl] >= HALF, PF, 0)
            for b in range(16):
                row = g * 16 + b
                uo = ucolv[b]
                io = icolv[b]
                p = (u_chunk[row, pl.ds(uo, 16)]
                     * v_chunk[row, pl.ds(io, 16)]) * w_chunks[0]
                for cc in range(1, PF // 16):
                    uc = u_chunk[row, pl.ds(uo + cc * 16, 16)]
                    vc = v_chunk[row, pl.ds(io + cc * 16, 16)]
                    p = p + (uc * vc) * w_chunks[cc]
                plsc.store_scatter(t_v, [lanes * 16 + b], p)
            acc = t_v[pl.ds(0, 16)]
            for l in range(1, 16):
                acc = acc + t_v[pl.ds(l * 16, 16)]
            z = acc + bias
            out_v[pl.ds(c * CHUNK + g * 16, 16)] = 1.0 / (1.0 + jnp.exp(-z))
            return carry

        lax.fori_loop(0, GROUPS_PER_CHUNK, group_body, 0, unroll=False)

    pltpu.sync_copy(out_v, out_hbm.at[pl.ds(base, B_PER_W)])


@jax.jit
def _gmf(uid, iid, utT, itT, w, b16):
    ut2, it2 = _transpose_tables(utT, itT)
    mesh = plsc.VectorSubcoreMesh(
        core_axis_name="c", subcore_axis_name="s", num_cores=NC, num_subcores=NS)
    fn = pl.kernel(
        _sc_gmf_body,
        out_type=jax.ShapeDtypeStruct((BATCH,), jnp.float32),
        mesh=mesh,
        scratch_types=[
            pltpu.VMEM((N_CHUNKS, CHUNK), jnp.int32),
            pltpu.VMEM((N_CHUNKS, CHUNK), jnp.int32),
            pltpu.VMEM((N_CHUNKS, CHUNK), jnp.int32),
            pltpu.VMEM((N_CHUNKS, CHUNK), jnp.int32),
            pltpu.VMEM((2, CHUNK, 2 * PF), jnp.float32),
            pltpu.VMEM((2, CHUNK, 2 * PF), jnp.float32),
            pltpu.VMEM((PF,), jnp.float32),
            pltpu.VMEM((16,), jnp.float32),
            pltpu.VMEM((B_PER_W,), jnp.float32),
            pltpu.VMEM((256,), jnp.float32),
            pltpu.SemaphoreType.DMA,
            pltpu.SemaphoreType.DMA,
        ],
        compiler_params=pltpu.CompilerParams(needs_layout_passes=False),
    )
    return fn(uid, iid, ut2, it2, w, b16)


def kernel(x, user_table, item_table, W_out, b_out):
    uid = x[:, 0].astype(jnp.int32)
    iid = x[:, 1].astype(jnp.int32)
    utT = user_table.T
    itT = item_table.T
    w = W_out.reshape(-1).astype(jnp.float32)
    b16 = jnp.broadcast_to(b_out.reshape(-1), (16,)).astype(jnp.float32)
    return _gmf(uid, iid, utT, itT, w, b16)


# speedup vs baseline: 1.0403x; 1.0403x over previous
import functools

import jax
import jax.numpy as jnp
from jax import lax
from jax.experimental import pallas as pl
from jax.experimental.pallas import tpu as pltpu
from jax.experimental.pallas import tpu_sc as plsc

BATCH = 16384
PF = 64
NC = 2
NS = 16
NW = NC * NS
B_PER_W = BATCH // NW
CHUNK = 128
N_CHUNKS = B_PER_W // CHUNK
GROUPS_PER_CHUNK = CHUNK // 16

TR_COLS = 1024
HALF_BLOCKS = 49
HALF = HALF_BLOCKS * TR_COLS
PAIRS = 4
TR_GRID = 13


def _tr_body(*refs):
    ua, ub = refs[0:PAIRS], refs[PAIRS:2 * PAIRS]
    ia, ib = refs[2 * PAIRS:3 * PAIRS], refs[3 * PAIRS:4 * PAIRS]
    uo_ref, io_ref = refs[4 * PAIRS], refs[4 * PAIRS + 1]
    for p in range(PAIRS):
        sl = pl.ds(p * TR_COLS, TR_COLS)
        uo_ref[sl, :] = jnp.concatenate([ua[p][...], ub[p][...]], axis=0).T
        io_ref[sl, :] = jnp.concatenate([ia[p][...], ib[p][...]], axis=0).T


def _transpose_tables(utT, itT):
    def a_spec(p):
        return pl.BlockSpec(
            (PF, TR_COLS),
            lambda i, p=p: (0, jnp.minimum(PAIRS * i + p, 2 * HALF_BLOCKS - 1)))

    def b_spec(p):
        return pl.BlockSpec(
            (PF, TR_COLS),
            lambda i, p=p: (0, jnp.minimum(HALF_BLOCKS + PAIRS * i + p,
                                           2 * HALF_BLOCKS - 1)))

    specs = [a_spec(p) for p in range(PAIRS)] + [b_spec(p) for p in range(PAIRS)]
    out_sds = jax.ShapeDtypeStruct((TR_GRID * PAIRS * TR_COLS, 128), jnp.float32)
    return pl.pallas_call(
        _tr_body,
        grid=(TR_GRID,),
        in_specs=specs + specs,
        out_specs=[pl.BlockSpec((PAIRS * TR_COLS, 128), lambda i: (i, 0))] * 2,
        out_shape=[out_sds, out_sds],
    )(*([utT] * (2 * PAIRS) + [itT] * (2 * PAIRS)))


def _sc_gmf_body(uid_hbm, iid_hbm, ut_hbm, it_hbm, w_hbm, b_hbm, out_hbm,
                 uidx_v, iidx_v, usup_v, isup_v, u_chunks, v_chunks,
                 w_v, b_v, out_v, t_v, idx_sem, sem):
    wid = lax.axis_index("s") * NC + lax.axis_index("c")
    base = wid * B_PER_W

    idx_cps = []
    for c in range(N_CHUNKS):
        idx_cps.append(pltpu.async_copy(
            uid_hbm.at[pl.ds(base + c * CHUNK, CHUNK)], uidx_v.at[c], idx_sem))
        idx_cps.append(pltpu.async_copy(
            iid_hbm.at[pl.ds(base + c * CHUNK, CHUNK)], iidx_v.at[c], idx_sem))
    idx_cps.append(pltpu.async_copy(w_hbm, w_v, idx_sem))
    idx_cps.append(pltpu.async_copy(b_hbm, b_v, idx_sem))
    for cp in idx_cps:
        cp.wait()

    for c in range(N_CHUNKS):
        for k in range(CHUNK // 16):
            sl = pl.ds(k * 16, 16)
            uv = uidx_v[c, sl]
            iv = iidx_v[c, sl]
            usup_v[c, sl] = uv - jnp.where(uv >= HALF, HALF, 0)
            isup_v[c, sl] = iv - jnp.where(iv >= HALF, HALF, 0)

    bias = b_v[...][0]
    w_chunks = [w_v[pl.ds(c * 16, 16)] for c in range(PF // 16)]
    lanes = lax.iota(jnp.int32, 16)

    cps = [None] * N_CHUNKS

    SUB = 32

    def fire(c):
        buf = c % 2
        cp = []
        for k in range(CHUNK // SUB):
            sl = pl.ds(k * SUB, SUB)
            cp.append(pltpu.async_copy(
                ut_hbm.at[usup_v.at[c, sl]], u_chunks.at[buf, sl], sem))
            cp.append(pltpu.async_copy(
                it_hbm.at[isup_v.at[c, sl]], v_chunks.at[buf, sl], sem))
        cps[c] = cp

    fire(0)
    for c in range(N_CHUNKS):
        if c + 1 < N_CHUNKS:
            fire(c + 1)
        for cp in cps[c]:
            cp.wait()
        u_chunk = u_chunks.at[c % 2]
        v_chunk = v_chunks.at[c % 2]

        def group_body(g, carry, c=c, u_chunk=u_chunk, v_chunk=v_chunk):
            sl = pl.ds(g * 16, 16)
            ucolv = jnp.where(uidx_v[c, sl] >= HALF, PF, 0)
            icolv = jnp.where(iidx_v[c, sl] >= HALF, PF, 0)
            for b in range(16):
                row = g * 16 + b
                uo = ucolv[b]
                io = icolv[b]
                p = (u_chunk[row, pl.ds(uo, 16)]
                     * v_chunk[row, pl.ds(io, 16)]) * w_chunks[0]
                for cc in range(1, PF // 16):
                    uc = u_chunk[row, pl.ds(uo + cc * 16, 16)]
                    vc = v_chunk[row, pl.ds(io + cc * 16, 16)]
                    p = p + (uc * vc) * w_chunks[cc]
                plsc.store_scatter(t_v, [lanes * 16 + b], p)
            acc = t_v[pl.ds(0, 16)]
            for l in range(1, 16):
                acc = acc + t_v[pl.ds(l * 16, 16)]
            z = acc + bias
            out_v[pl.ds(c * CHUNK + g * 16, 16)] = 1.0 / (1.0 + jnp.exp(-z))
            return carry

        lax.fori_loop(0, GROUPS_PER_CHUNK, group_body, 0, unroll=False)

    pltpu.sync_copy(out_v, out_hbm.at[pl.ds(base, B_PER_W)])


@jax.jit
def _gmf(uid, iid, utT, itT, w, b16):
    ut2, it2 = _transpose_tables(utT, itT)
    mesh = plsc.VectorSubcoreMesh(
        core_axis_name="c", subcore_axis_name="s", num_cores=NC, num_subcores=NS)
    fn = pl.kernel(
        _sc_gmf_body,
        out_type=jax.ShapeDtypeStruct((BATCH,), jnp.float32),
        mesh=mesh,
        scratch_types=[
            pltpu.VMEM((N_CHUNKS, CHUNK), jnp.int32),
            pltpu.VMEM((N_CHUNKS, CHUNK), jnp.int32),
            pltpu.VMEM((N_CHUNKS, CHUNK), jnp.int32),
            pltpu.VMEM((N_CHUNKS, CHUNK), jnp.int32),
            pltpu.VMEM((2, CHUNK, 2 * PF), jnp.float32),
            pltpu.VMEM((2, CHUNK, 2 * PF), jnp.float32),
            pltpu.VMEM((PF,), jnp.float32),
            pltpu.VMEM((16,), jnp.float32),
            pltpu.VMEM((B_PER_W,), jnp.float32),
            pltpu.VMEM((256,), jnp.float32),
            pltpu.SemaphoreType.DMA,
            pltpu.SemaphoreType.DMA,
        ],
        compiler_params=pltpu.CompilerParams(needs_layout_passes=False),
    )
    return fn(uid, iid, ut2, it2, w, b16)


def kernel(x, user_table, item_table, W_out, b_out):
    uid = x[:, 0].astype(jnp.int32)
    iid = x[:, 1].astype(jnp.int32)
    utT = user_table.T
    itT = item_table.T
    w = W_out.reshape(-1).astype(jnp.float32)
    b16 = jnp.broadcast_to(b_out.reshape(-1), (16,)).astype(jnp.float32)
    return _gmf(uid, iid, utT, itT, w, b16)
